# baseline (speedup 1.0000x reference)
import dataclasses
import functools

import jax
import jax.numpy as jnp
from jax import lax
from jax.experimental import pallas as pl
from jax.experimental.pallas import tpu as pltpu
from jax.experimental.pallas import tpu_sc as plsc

NLAB = 33
LANES = 16
NLAB_PAD = 48
ACC_PAD = NLAB_PAD * LANES
NW = 32
ROWS = 51200
MINOR = 160
BROWS = 40
BROWS2 = 128
BLK = BROWS * MINOR
PROW = 2 * NLAB_PAD

_MESH = plsc.VectorSubcoreMesh(core_axis_name="c", subcore_axis_name="s")

_CP = pltpu.CompilerParams(use_tc_tiling_on_sc=True)
if "needs_layout_passes" in pltpu.CompilerParams.__dataclass_fields__:
    _CP = dataclasses.replace(_CP, needs_layout_passes=False)


def _pass1_body(lm_hbm, n_hbm, g_hbm, m_hbm, mu_hbm, sg_hbm,
                pk_hbm, part_hbm, mu_v, sg_v, sums_v, cnts_v, prow_v):
    nblk = lm_hbm.shape[0] // BROWS
    wid = lax.axis_index("s") * 2 + lax.axis_index("c")
    pltpu.sync_copy(mu_hbm, mu_v)
    pltpu.sync_copy(sg_hbm, sg_v)

    zero = jnp.zeros((LANES,), jnp.float32)

    @pl.loop(0, ACC_PAD, step=LANES)
    def _(i):
        sums_v[pl.ds(i, LANES)] = zero
        cnts_v[pl.ds(i, LANES)] = zero

    lane = lax.iota(jnp.int32, LANES)
    one = jnp.ones((LANES,), jnp.float32)

    def body(lm_b, n_b, g_b, m_b, pk_b):
        @plsc.parallel_loop(0, BROWS)
        def _(r):
            for v in range(MINOR // LANES):
                sl = (r, pl.ds(v * LANES, LANES))
                lm16 = lm_b[sl]
                muv = plsc.load_gather(mu_v, [lm16])
                sgv = plsc.load_gather(sg_v, [lm16])
                tex = (muv + sgv * n_b[sl]) \
                    * (0.5 + g_b[sl]) * (0.1 + 0.65 * m_b[sl])
                tbits = plsc.bitcast(tex, jnp.int32)
                tbits = (tbits + 0x8000) & jnp.int32(-65536)
                pk_b[pl.ds(r * MINOR + v * LANES, LANES)] = tbits | lm16
                idx = lm16 * LANES + lane
                plsc.addupdate_scatter(sums_v, [idx], tex)
                plsc.addupdate_scatter(cnts_v, [idx], one)

    spec2d = pl.BlockSpec(block_shape=(BROWS, MINOR), index_map=lambda i: (i, 0))
    spec1d = pl.BlockSpec(block_shape=(BLK,), index_map=lambda i: (i,))
    pltpu.emit_pipeline(
        body,
        grid=(nblk,),
        in_specs=[spec2d] * 4,
        out_specs=[spec1d],
        core_axis_name=("c", "s"),
        dimension_semantics=(pltpu.PARALLEL,),
    )(lm_hbm, n_hbm, g_hbm, m_hbm, pk_hbm)

    for j in range(NLAB_PAD // LANES):
        labv = lax.iota(jnp.int32, LANES) + j * LANES
        tot_s = zero
        tot_c = zero
        for k in range(LANES):
            idx = labv * LANES + k
            tot_s = tot_s + plsc.load_gather(sums_v, [idx])
            tot_c = tot_c + plsc.load_gather(cnts_v, [idx])
        prow_v[pl.ds(j * LANES, LANES)] = tot_s
        prow_v[pl.ds(NLAB_PAD + j * LANES, LANES)] = tot_c

    pltpu.sync_copy(prow_v, part_hbm.at[pl.ds(wid * PROW, PROW)])


def _pass2_body(part_hbm, inten_hbm, pk_hbm, out_hbm,
                part_v, inten_v, shift_v):
    nblk = out_hbm.shape[0] // BROWS2
    pltpu.sync_copy(part_hbm, part_v)
    pltpu.sync_copy(inten_hbm, inten_v)

    for j in range(NLAB_PAD // LANES):
        def wbody(w, sc, j=j):
            s, c = sc
            s = s + part_v[pl.ds(w * PROW + j * LANES, LANES)]
            c = c + part_v[pl.ds(w * PROW + NLAB_PAD + j * LANES, LANES)]
            return (s, c)

        s, c = lax.fori_loop(
            0, NW, wbody,
            (jnp.zeros((LANES,), jnp.float32), jnp.zeros((LANES,), jnp.float32)))
        mean = s / jnp.maximum(c, 1.0)
        shift = mean - inten_v[pl.ds(j * LANES, LANES)]
        labv = lax.iota(jnp.int32, LANES) + j * LANES
        shift = jnp.where((labv > 0) & (labv < NLAB), shift, 0.0)
        shift_v[pl.ds(j * LANES, LANES)] = shift

    def body(pk_b, out_b):
        @plsc.parallel_loop(0, BROWS2)
        def _(r):
            for v in range(MINOR // LANES):
                sl = (r, pl.ds(v * LANES, LANES))
                pk = pk_b[pl.ds(r * MINOR + v * LANES, LANES)]
                lm16 = pk & 0xFF
                sv = plsc.load_gather(shift_v, [lm16])
                tex = plsc.bitcast(pk & jnp.int32(-65536), jnp.float32)
                out_b[sl] = tex - sv

    spec2d = pl.BlockSpec(block_shape=(BROWS2, MINOR), index_map=lambda i: (i, 0))
    spec1d = pl.BlockSpec(block_shape=(BROWS2 * MINOR,), index_map=lambda i: (i,))
    pltpu.emit_pipeline(
        body,
        grid=(nblk,),
        in_specs=[spec1d],
        out_specs=[spec2d],
        core_axis_name=("c", "s"),
        dimension_semantics=(pltpu.PARALLEL,),
    )(pk_hbm, out_hbm)


def kernel(label_map, noise, gamma_noise, mul_field, mu, sigma_tbl, intensity_vals):
    shape = label_map.shape
    n = label_map.size
    assert n == ROWS * MINOR

    lm = label_map.astype(jnp.int32).reshape(ROWS, MINOR)
    nz = noise.reshape(ROWS, MINOR)
    gm = gamma_noise.reshape(ROWS, MINOR)
    mf = mul_field.reshape(ROWS, MINOR)
    pad = (0, NLAB_PAD - NLAB)
    mu_p = jnp.pad(mu, pad)
    sg_p = jnp.pad(sigma_tbl, pad)
    in_p = jnp.pad(intensity_vals, pad)

    pass1 = pl.kernel(
        _pass1_body,
        out_type=[
            jax.ShapeDtypeStruct((ROWS * MINOR,), jnp.int32),
            jax.ShapeDtypeStruct((NW * PROW,), jnp.float32),
        ],
        mesh=_MESH,
        scratch_types=[
            pltpu.VMEM((NLAB_PAD,), jnp.float32),
            pltpu.VMEM((NLAB_PAD,), jnp.float32),
            pltpu.VMEM((ACC_PAD,), jnp.float32),
            pltpu.VMEM((ACC_PAD,), jnp.float32),
            pltpu.VMEM((PROW,), jnp.float32),
        ],
        compiler_params=_CP,
    )
    pk, part = pass1(lm, nz, gm, mf, mu_p, sg_p)

    pass2 = pl.kernel(
        _pass2_body,
        out_type=jax.ShapeDtypeStruct((ROWS, MINOR), jnp.float32),
        mesh=_MESH,
        scratch_types=[
            pltpu.VMEM((NW * PROW,), jnp.float32),
            pltpu.VMEM((NLAB_PAD,), jnp.float32),
            pltpu.VMEM((NLAB_PAD,), jnp.float32),
        ],
        compiler_params=_CP,
    )
    out = pass2(part, in_p, pk)
    return out.reshape(shape)


# speedup vs baseline: 1.0348x; 1.0348x over previous
import dataclasses
import functools

import jax
import jax.numpy as jnp
from jax import lax
from jax.experimental import pallas as pl
from jax.experimental.pallas import tpu as pltpu
from jax.experimental.pallas import tpu_sc as plsc

NLAB = 33
LANES = 16
NLAB_PAD = 48
ACC_PAD = NLAB_PAD * LANES
NW = 32
ROWS = 51200
MINOR = 160
BROWS = 40
BROWS2 = 128
BLK = BROWS * MINOR
PROW = 2 * NLAB_PAD

_MESH = plsc.VectorSubcoreMesh(core_axis_name="c", subcore_axis_name="s")

_CP = pltpu.CompilerParams(use_tc_tiling_on_sc=True)
if "needs_layout_passes" in pltpu.CompilerParams.__dataclass_fields__:
    _CP = dataclasses.replace(_CP, needs_layout_passes=False)


def _pass1_body(lm_hbm, n_hbm, g_hbm, m_hbm, tbl_hbm,
                pk_hbm, part_hbm, tbl_v, sums_v, cnts_v, prow_v):
    nblk = lm_hbm.shape[0] // BROWS
    wid = lax.axis_index("s") * 2 + lax.axis_index("c")
    pltpu.sync_copy(tbl_hbm, tbl_v)

    zero = jnp.zeros((LANES,), jnp.float32)

    @pl.loop(0, ACC_PAD, step=LANES)
    def _(i):
        sums_v[pl.ds(i, LANES)] = zero
        cnts_v[pl.ds(i, LANES)] = zero

    lane = lax.iota(jnp.int32, LANES)
    one = jnp.ones((LANES,), jnp.float32)

    def body(lm_b, n_b, g_b, m_b, pk_b):
        @plsc.parallel_loop(0, BROWS)
        def _(r):
            for v in range(MINOR // LANES):
                sl = (r, pl.ds(v * LANES, LANES))
                lm16 = lm_b[sl]
                ms = plsc.load_gather(tbl_v, [lm16])
                muv = plsc.bitcast(ms & jnp.int32(-65536), jnp.float32)
                sgv = plsc.bitcast(ms << 16, jnp.float32)
                tex = (muv + sgv * n_b[sl]) \
                    * (0.5 + g_b[sl]) * (0.1 + 0.65 * m_b[sl])
                tbits = plsc.bitcast(tex, jnp.int32)
                tbits = (tbits + 0x8000) & jnp.int32(-65536)
                pk_b[pl.ds(r * MINOR + v * LANES, LANES)] = tbits | lm16
                idx = lm16 * LANES + lane
                plsc.addupdate_scatter(sums_v, [idx], tex)
                plsc.addupdate_scatter(cnts_v, [idx], one)

    spec2d = pl.BlockSpec(block_shape=(BROWS, MINOR), index_map=lambda i: (i, 0))
    spec1d = pl.BlockSpec(block_shape=(BLK,), index_map=lambda i: (i,))
    pltpu.emit_pipeline(
        body,
        grid=(nblk,),
        in_specs=[spec2d] * 4,
        out_specs=[spec1d],
        core_axis_name=("c", "s"),
        dimension_semantics=(pltpu.PARALLEL,),
    )(lm_hbm, n_hbm, g_hbm, m_hbm, pk_hbm)

    for j in range(NLAB_PAD // LANES):
        labv = lax.iota(jnp.int32, LANES) + j * LANES
        tot_s = zero
        tot_c = zero
        for k in range(LANES):
            idx = labv * LANES + k
            tot_s = tot_s + plsc.load_gather(sums_v, [idx])
            tot_c = tot_c + plsc.load_gather(cnts_v, [idx])
        prow_v[pl.ds(j * LANES, LANES)] = tot_s
        prow_v[pl.ds(NLAB_PAD + j * LANES, LANES)] = tot_c

    pltpu.sync_copy(prow_v, part_hbm.at[pl.ds(wid * PROW, PROW)])


def _pass2_body(part_hbm, inten_hbm, pk_hbm, out_hbm,
                part_v, inten_v, shift_v):
    nblk = out_hbm.shape[0] // BROWS2
    pltpu.sync_copy(part_hbm, part_v)
    pltpu.sync_copy(inten_hbm, inten_v)

    for j in range(NLAB_PAD // LANES):
        def wbody(w, sc, j=j):
            s, c = sc
            s = s + part_v[pl.ds(w * PROW + j * LANES, LANES)]
            c = c + part_v[pl.ds(w * PROW + NLAB_PAD + j * LANES, LANES)]
            return (s, c)

        s, c = lax.fori_loop(
            0, NW, wbody,
            (jnp.zeros((LANES,), jnp.float32), jnp.zeros((LANES,), jnp.float32)))
        mean = s / jnp.maximum(c, 1.0)
        shift = mean - inten_v[pl.ds(j * LANES, LANES)]
        labv = lax.iota(jnp.int32, LANES) + j * LANES
        shift = jnp.where((labv > 0) & (labv < NLAB), shift, 0.0)
        shift_v[pl.ds(j * LANES, LANES)] = shift

    def body(pk_b, out_b):
        @plsc.parallel_loop(0, BROWS2)
        def _(r):
            for v in range(MINOR // LANES):
                sl = (r, pl.ds(v * LANES, LANES))
                pk = pk_b[pl.ds(r * MINOR + v * LANES, LANES)]
                lm16 = pk & 0xFF
                sv = plsc.load_gather(shift_v, [lm16])
                tex = plsc.bitcast(pk & jnp.int32(-65536), jnp.float32)
                out_b[sl] = tex - sv

    spec2d = pl.BlockSpec(block_shape=(BROWS2, MINOR), index_map=lambda i: (i, 0))
    spec1d = pl.BlockSpec(block_shape=(BROWS2 * MINOR,), index_map=lambda i: (i,))
    pltpu.emit_pipeline(
        body,
        grid=(nblk,),
        in_specs=[spec1d],
        out_specs=[spec2d],
        core_axis_name=("c", "s"),
        dimension_semantics=(pltpu.PARALLEL,),
    )(pk_hbm, out_hbm)


def kernel(label_map, noise, gamma_noise, mul_field, mu, sigma_tbl, intensity_vals):
    shape = label_map.shape
    n = label_map.size
    assert n == ROWS * MINOR

    lm = label_map.astype(jnp.int32).reshape(ROWS, MINOR)
    nz = noise.reshape(ROWS, MINOR)
    gm = gamma_noise.reshape(ROWS, MINOR)
    mf = mul_field.reshape(ROWS, MINOR)
    pad = (0, NLAB_PAD - NLAB)
    mu_b = lax.bitcast_convert_type(
        jnp.pad(mu, pad).astype(jnp.bfloat16), jnp.uint16).astype(jnp.uint32)
    sg_b = lax.bitcast_convert_type(
        jnp.pad(sigma_tbl, pad).astype(jnp.bfloat16), jnp.uint16).astype(jnp.uint32)
    tbl_p = ((mu_b << 16) | sg_b).astype(jnp.int32)
    in_p = jnp.pad(intensity_vals, pad)

    pass1 = pl.kernel(
        _pass1_body,
        out_type=[
            jax.ShapeDtypeStruct((ROWS * MINOR,), jnp.int32),
            jax.ShapeDtypeStruct((NW * PROW,), jnp.float32),
        ],
        mesh=_MESH,
        scratch_types=[
            pltpu.VMEM((NLAB_PAD,), jnp.int32),
            pltpu.VMEM((ACC_PAD,), jnp.float32),
            pltpu.VMEM((ACC_PAD,), jnp.float32),
            pltpu.VMEM((PROW,), jnp.float32),
        ],
        compiler_params=_CP,
    )
    pk, part = pass1(lm, nz, gm, mf, tbl_p)

    pass2 = pl.kernel(
        _pass2_body,
        out_type=jax.ShapeDtypeStruct((ROWS, MINOR), jnp.float32),
        mesh=_MESH,
        scratch_types=[
            pltpu.VMEM((NW * PROW,), jnp.float32),
            pltpu.VMEM((NLAB_PAD,), jnp.float32),
            pltpu.VMEM((NLAB_PAD,), jnp.float32),
        ],
        compiler_params=_CP,
    )
    out = pass2(part, in_p, pk)
    return out.reshape(shape)


# speedup vs baseline: 1.0380x; 1.0031x over previous
import dataclasses

import jax
import jax.numpy as jnp
from jax import lax
from jax.experimental import pallas as pl
from jax.experimental.pallas import tpu as pltpu
from jax.experimental.pallas import tpu_sc as plsc

NLAB = 33
LANES = 16
NLAB_PAD = 48
ACC_PAD = NLAB_PAD * LANES
NW = 32
ROWS = 51200
MINOR = 160
BROWS = 40
BROWS2 = 128
BLK = BROWS * MINOR
PROW = 2 * NLAB_PAD

_MESH = plsc.VectorSubcoreMesh(core_axis_name="c", subcore_axis_name="s")

_CP = pltpu.CompilerParams(use_tc_tiling_on_sc=True)
if "needs_layout_passes" in pltpu.CompilerParams.__dataclass_fields__:
    _CP = dataclasses.replace(_CP, needs_layout_passes=False)


def _pass1_body(lm_hbm, n_hbm, g_hbm, m_hbm, tbl_hbm,
                pk_hbm, part_hbm, tbl_v, sums_v, cnts_v, prow_v):
    nblk = lm_hbm.shape[0] // BROWS
    wid = lax.axis_index("s") * 2 + lax.axis_index("c")
    pltpu.sync_copy(tbl_hbm, tbl_v)

    zero = jnp.zeros((LANES,), jnp.float32)

    @pl.loop(0, ACC_PAD, step=LANES)
    def _(i):
        sums_v[pl.ds(i, LANES)] = zero
        cnts_v[pl.ds(i, LANES)] = zero

    lane = lax.iota(jnp.int32, LANES)
    one = jnp.ones((LANES,), jnp.float32)

    def body(lm_b, n_b, g_b, m_b, pk_b):
        @plsc.parallel_loop(0, BROWS)
        def _(r):
            for v in range(MINOR // LANES):
                sl = (r, pl.ds(v * LANES, LANES))
                lm16 = lm_b[sl]
                ms = plsc.load_gather(tbl_v, [lm16])
                muv = plsc.bitcast(ms & jnp.int32(-65536), jnp.float32)
                sgv = plsc.bitcast(ms << 16, jnp.float32)
                tex = (muv + sgv * n_b[sl]) \
                    * (0.5 + g_b[sl]) * (0.1 + 0.65 * m_b[sl])
                tbits = plsc.bitcast(tex, jnp.int32)
                tbits = (tbits + 0x8000) & jnp.int32(-65536)
                pk_b[pl.ds(r * MINOR + v * LANES, LANES)] = tbits | lm16
                idx = lm16 * LANES + lane
                plsc.addupdate_scatter(sums_v, [idx], tex)
                plsc.addupdate_scatter(cnts_v, [idx], one)

    spec2d = pl.BlockSpec(block_shape=(BROWS, MINOR), index_map=lambda i: (i, 0))
    spec1d = pl.BlockSpec(block_shape=(BLK,), index_map=lambda i: (i,))
    pltpu.emit_pipeline(
        body,
        grid=(nblk,),
        in_specs=[spec2d] * 4,
        out_specs=[spec1d],
        core_axis_name=("c", "s"),
        dimension_semantics=(pltpu.PARALLEL,),
    )(lm_hbm, n_hbm, g_hbm, m_hbm, pk_hbm)

    for j in range(NLAB_PAD // LANES):
        labv = lax.iota(jnp.int32, LANES) + j * LANES
        tot_s = zero
        tot_c = zero
        for k in range(LANES):
            idx = labv * LANES + k
            tot_s = tot_s + plsc.load_gather(sums_v, [idx])
            tot_c = tot_c + plsc.load_gather(cnts_v, [idx])
        prow_v[pl.ds(j * LANES, LANES)] = tot_s
        prow_v[pl.ds(NLAB_PAD + j * LANES, LANES)] = tot_c

    pltpu.sync_copy(prow_v, part_hbm.at[pl.ds(wid * PROW, PROW)])


def _pass2_body(part_hbm, inten_hbm, pk_hbm, out_hbm,
                part_v, inten_v, shift_v):
    nblk = out_hbm.shape[0] // BROWS2
    pltpu.sync_copy(part_hbm, part_v)
    pltpu.sync_copy(inten_hbm, inten_v)

    for j in range(NLAB_PAD // LANES):
        def wbody(w, sc, j=j):
            s, c = sc
            s = s + part_v[pl.ds(w * PROW + j * LANES, LANES)]
            c = c + part_v[pl.ds(w * PROW + NLAB_PAD + j * LANES, LANES)]
            return (s, c)

        s, c = lax.fori_loop(
            0, NW, wbody,
            (jnp.zeros((LANES,), jnp.float32), jnp.zeros((LANES,), jnp.float32)))
        mean = s / jnp.maximum(c, 1.0)
        shift = mean - inten_v[pl.ds(j * LANES, LANES)]
        labv = lax.iota(jnp.int32, LANES) + j * LANES
        shift = jnp.where((labv > 0) & (labv < NLAB), shift, 0.0)
        shift_v[pl.ds(j * LANES, LANES)] = shift

    def body(pk_b, out_b):
        @plsc.parallel_loop(0, BROWS2)
        def _(r):
            for v in range(MINOR // LANES):
                sl = (r, pl.ds(v * LANES, LANES))
                pk = pk_b[pl.ds(r * MINOR + v * LANES, LANES)]
                lm16 = pk & 0xFF
                sv = plsc.load_gather(shift_v, [lm16])
                tex = plsc.bitcast(pk & jnp.int32(-65536), jnp.float32)
                out_b[sl] = tex - sv

    spec2d = pl.BlockSpec(block_shape=(BROWS2, MINOR), index_map=lambda i: (i, 0))
    spec1d = pl.BlockSpec(block_shape=(BROWS2 * MINOR,), index_map=lambda i: (i,))
    pltpu.emit_pipeline(
        body,
        grid=(nblk,),
        in_specs=[spec1d],
        out_specs=[spec2d],
        core_axis_name=("c", "s"),
        dimension_semantics=(pltpu.PARALLEL,),
    )(pk_hbm, out_hbm)


def kernel(label_map, noise, gamma_noise, mul_field, mu, sigma_tbl, intensity_vals):
    shape = label_map.shape
    n = label_map.size
    assert n == ROWS * MINOR

    lm = label_map.astype(jnp.int32).reshape(ROWS, MINOR)
    nz = noise.reshape(ROWS, MINOR)
    gm = gamma_noise.reshape(ROWS, MINOR)
    mf = mul_field.reshape(ROWS, MINOR)
    pad = (0, NLAB_PAD - NLAB)
    mu_b = lax.bitcast_convert_type(
        jnp.pad(mu, pad).astype(jnp.bfloat16), jnp.uint16).astype(jnp.uint32)
    sg_b = lax.bitcast_convert_type(
        jnp.pad(sigma_tbl, pad).astype(jnp.bfloat16), jnp.uint16).astype(jnp.uint32)
    tbl_p = ((mu_b << 16) | sg_b).astype(jnp.int32)
    in_p = jnp.pad(intensity_vals, pad)

    pass1 = pl.kernel(
        _pass1_body,
        out_type=[
            jax.ShapeDtypeStruct((ROWS * MINOR,), jnp.int32),
            jax.ShapeDtypeStruct((NW * PROW,), jnp.float32),
        ],
        mesh=_MESH,
        scratch_types=[
            pltpu.VMEM((NLAB_PAD,), jnp.int32),
            pltpu.VMEM((ACC_PAD,), jnp.float32),
            pltpu.VMEM((ACC_PAD,), jnp.float32),
            pltpu.VMEM((PROW,), jnp.float32),
        ],
        compiler_params=_CP,
    )
    pk, part = pass1(lm, nz, gm, mf, tbl_p)

    pass2 = pl.kernel(
        _pass2_body,
        out_type=jax.ShapeDtypeStruct((ROWS, MINOR), jnp.float32),
        mesh=_MESH,
        scratch_types=[
            pltpu.VMEM((NW * PROW,), jnp.float32),
            pltpu.VMEM((NLAB_PAD,), jnp.float32),
            pltpu.VMEM((NLAB_PAD,), jnp.float32),
        ],
        compiler_params=_CP,
    )
    out = pass2(part, in_p, pk)
    return out.reshape(shape)
